# baseline (speedup 1.0000x reference)
import functools

import jax
import jax.numpy as jnp
from jax.experimental import pallas as pl
from jax.experimental.pallas import tpu as pltpu


def _fused_body(x_ref, w1_ref, w2_ref, wf1_ref, wf2_ref, wf3_ref,
                bias_ref, out_ref, *, bt):
    f32 = jnp.float32
    bf16 = jnp.bfloat16
    b1 = bias_ref[0:1, :]
    b2 = bias_ref[1:2, :]

    A = x_ref[0].astype(bf16).reshape(bt, 3, 8, 4, 32)
    T = jnp.transpose(A, (2, 0, 3, 1, 4)).reshape(8, bt, 4, 96)
    T = jnp.concatenate([T, jnp.zeros((8, bt, 4, 32), bf16)], axis=-1)
    X2 = T.reshape(8 * bt, 512)

    def c1(r):
        ka = (4 - r) * 128
        acc = jnp.dot(X2[: 7 * bt, r * 128:], w1_ref[:ka, :],
                      preferred_element_type=f32)
        return acc + jnp.dot(X2[bt:, : (r + 1) * 128], w1_ref[ka:, :],
                             preferred_element_type=f32)

    def pool1(a, b):
        m = jnp.maximum(a, b)
        return jnp.maximum(jnp.maximum(m[:, :128], m[:, 128:]) + b1, 0.0)

    p_even = pool1(c1(0), c1(1))
    p_odd = pool1(c1(2), c1(3))
    p1 = jnp.concatenate([p_even, p_odd], axis=1).astype(bf16)

    y2e = (jnp.dot(p1[: 5 * bt, :], w2_ref[:256, :], preferred_element_type=f32)
           + jnp.dot(p1[bt:6 * bt, :], w2_ref[256:512, :], preferred_element_type=f32)
           + jnp.dot(p1[2 * bt:, :128], w2_ref[512:, :], preferred_element_type=f32))
    y2o = (jnp.dot(p1[: 5 * bt, 128:], w2_ref[:128, :], preferred_element_type=f32)
           + jnp.dot(p1[bt:6 * bt, :], w2_ref[128:384, :], preferred_element_type=f32)
           + jnp.dot(p1[2 * bt:, :], w2_ref[384:, :], preferred_element_type=f32))
    m2 = jnp.maximum(y2e, y2o)
    p2 = jnp.maximum(jnp.maximum(m2[:, :128], m2[:, 128:]) + b2, 0.0).astype(bf16)

    f1in = jnp.concatenate([p2[q * bt:(q + 1) * bt, :] for q in range(5)], axis=1)
    f1 = jnp.dot(f1in, wf1_ref[...], preferred_element_type=f32) + bias_ref[2:3, :]
    f1 = jnp.maximum(f1, 0.0).astype(bf16)
    f2 = jnp.dot(f1, wf2_ref[...], preferred_element_type=f32) + bias_ref[3:4, :]
    f2 = jnp.maximum(f2, 0.0).astype(bf16)
    f3 = jnp.dot(f2, wf3_ref[...], preferred_element_type=f32) + bias_ref[4:5, :]
    out_ref[...] = f3.astype(out_ref.dtype)


def kernel(x, w1, w2, wf1, wf2, wf3, biases):
    N = x.shape[0]
    bt = 256
    while bt > 8 and (N + bt - 1) // bt < 2:
        bt //= 2
    n_pad = (-N) % bt
    nt = (N + n_pad) // bt

    xv = x
    if n_pad:
        xv = jnp.pad(xv, ((0, n_pad), (0, 0), (0, 0), (0, 0)))
    xv = xv.reshape(nt, bt, 3, 8, 128)

    w_idx = jnp.arange(32)
    c_idx = jnp.arange(4)
    perm = jnp.where(c_idx[:, None] < 3, w_idx[None, :] * 3 + c_idx[:, None],
                     96 + w_idx[None, :]).reshape(128)
    w1p = w1.reshape(5, 128, 256)[:, perm, :].reshape(640, 256)

    out = pl.pallas_call(
        functools.partial(_fused_body, bt=bt),
        out_shape=jax.ShapeDtypeStruct((nt * bt, 128), jnp.float32),
        grid=(nt,),
        in_specs=[
            pl.BlockSpec((1, bt, 3, 8, 128), lambda i: (i, 0, 0, 0, 0)),
            pl.BlockSpec((640, 256), lambda i: (0, 0)),
            pl.BlockSpec((640, 256), lambda i: (0, 0)),
            pl.BlockSpec((640, 128), lambda i: (0, 0)),
            pl.BlockSpec((128, 128), lambda i: (0, 0)),
            pl.BlockSpec((128, 128), lambda i: (0, 0)),
            pl.BlockSpec((8, 128), lambda i: (0, 0)),
        ],
        out_specs=pl.BlockSpec((bt, 128), lambda i: (i, 0)),
        compiler_params=pltpu.CompilerParams(
            dimension_semantics=("parallel",),
            vmem_limit_bytes=64 * 1024 * 1024,
        ),
    )(xv, w1p, w2, wf1, wf2, wf3, biases)

    return out[:N, :10]


# speedup vs baseline: 1.7271x; 1.7271x over previous
import functools

import jax
import jax.numpy as jnp
from jax.experimental import pallas as pl
from jax.experimental.pallas import tpu as pltpu


def _fused_body(*refs, bt):
    xb = [refs[3 * s:3 * s + 3] for s in range(8)]
    w1s_ref, w2_ref, wf1_ref, wf2_ref, wf3_ref, bias_ref, out_ref = refs[24:]
    f32 = jnp.float32
    bf16 = jnp.bfloat16
    b1 = bias_ref[0:1, :]
    b2 = bias_ref[1:2, :]

    X = [[xb[s][c][0].astype(bf16) for c in range(3)] for s in range(8)]

    def w1blk(r, part, c):
        m = (r * 2 + part) * 3 + c
        return w1s_ref[m * 128:(m + 1) * 128, :]

    def c1(r, s):
        acc = jnp.dot(X[s][0], w1blk(r, 0, 0), preferred_element_type=f32)
        for c in (1, 2):
            acc += jnp.dot(X[s][c], w1blk(r, 0, c), preferred_element_type=f32)
        for c in (0, 1, 2):
            acc += jnp.dot(X[s + 1][c], w1blk(r, 1, c), preferred_element_type=f32)
        return acc

    def pool1(a, b):
        m = jnp.maximum(a, b)
        return jnp.maximum(jnp.maximum(m[:, :128], m[:, 128:]) + b1, 0.0)

    p1 = []
    for s in range(7):
        pe = pool1(c1(0, s), c1(1, s))
        po = pool1(c1(2, s), c1(3, s))
        p1.append(jnp.concatenate([pe, po], axis=1).astype(bf16))

    p2 = []
    for q in range(5):
        ye = (jnp.dot(p1[q], w2_ref[:256, :], preferred_element_type=f32)
              + jnp.dot(p1[q + 1], w2_ref[256:512, :], preferred_element_type=f32)
              + jnp.dot(p1[q + 2][:, :128], w2_ref[512:, :], preferred_element_type=f32))
        yo = (jnp.dot(p1[q][:, 128:], w2_ref[:128, :], preferred_element_type=f32)
              + jnp.dot(p1[q + 1], w2_ref[128:384, :], preferred_element_type=f32)
              + jnp.dot(p1[q + 2], w2_ref[384:, :], preferred_element_type=f32))
        m2 = jnp.maximum(ye, yo)
        p2.append(jnp.maximum(jnp.maximum(m2[:, :128], m2[:, 128:]) + b2,
                              0.0).astype(bf16))

    f1in = jnp.concatenate(p2, axis=1)
    f1 = jnp.dot(f1in, wf1_ref[...], preferred_element_type=f32) + bias_ref[2:3, :]
    f1 = jnp.maximum(f1, 0.0).astype(bf16)
    f2 = jnp.dot(f1, wf2_ref[...], preferred_element_type=f32) + bias_ref[3:4, :]
    f2 = jnp.maximum(f2, 0.0).astype(bf16)
    f3 = jnp.dot(f2, wf3_ref[...], preferred_element_type=f32) + bias_ref[4:5, :]
    out_ref[...] = f3.astype(out_ref.dtype)


def _build_w1_blocks(w1):
    g = jnp.arange(4)[:, None]
    w = jnp.arange(32)[None, :]
    blocks = []
    for r in range(4):
        for part in range(2):
            for c in range(3):
                kh = g - r if part == 0 else g + 4 - r
                ok = jnp.broadcast_to((kh >= 0) & (kh <= 4), (4, 32))
                rows = (jnp.clip(kh, 0, 4) * 128 + w * 3 + c).reshape(128)
                blk = w1[rows, :] * ok.reshape(128, 1).astype(w1.dtype)
                blocks.append(blk)
    return jnp.concatenate(blocks, axis=0)


def kernel(x, w1, w2, wf1, wf2, wf3, biases):
    N = x.shape[0]
    bt = 256
    while bt > 8 and (N + bt - 1) // bt < 2:
        bt //= 2
    n_pad = (-N) % bt
    nt = (N + n_pad) // bt

    xv = x
    if n_pad:
        xv = jnp.pad(xv, ((0, n_pad), (0, 0), (0, 0), (0, 0)))
    xv = xv.reshape(nt, bt, 3072)

    w1s = _build_w1_blocks(w1)

    def xspec(c, s):
        return pl.BlockSpec((1, bt, 128),
                            lambda i, c=c, s=s: (i, 0, c * 8 + s))

    x_specs = [xspec(c, s) for s in range(8) for c in range(3)]
    const_specs = [
        pl.BlockSpec((3072, 256), lambda i: (0, 0)),
        pl.BlockSpec((640, 256), lambda i: (0, 0)),
        pl.BlockSpec((640, 128), lambda i: (0, 0)),
        pl.BlockSpec((128, 128), lambda i: (0, 0)),
        pl.BlockSpec((128, 128), lambda i: (0, 0)),
        pl.BlockSpec((8, 128), lambda i: (0, 0)),
    ]

    out = pl.pallas_call(
        functools.partial(_fused_body, bt=bt),
        out_shape=jax.ShapeDtypeStruct((nt * bt, 128), jnp.float32),
        grid=(nt,),
        in_specs=x_specs + const_specs,
        out_specs=pl.BlockSpec((bt, 128), lambda i: (i, 0)),
        compiler_params=pltpu.CompilerParams(
            dimension_semantics=("parallel",),
            vmem_limit_bytes=64 * 1024 * 1024,
        ),
    )(*([xv] * 24), w1s, w2, wf1, wf2, wf3, biases)

    return out[:N, :10]


# speedup vs baseline: 1.7363x; 1.0053x over previous
import functools

import jax
import jax.numpy as jnp
from jax.experimental import pallas as pl
from jax.experimental.pallas import tpu as pltpu


def _fused_body(x_ref, w1s_ref, w2_ref, wf1_ref, wf2_ref, wf3_ref,
                bias_ref, out_ref, *, bt):
    f32 = jnp.float32
    bf16 = jnp.bfloat16
    b1 = bias_ref[0:1, :]
    b2 = bias_ref[1:2, :]

    A = x_ref[0].astype(bf16)
    X = [[A[:, (c * 8 + s) * 128:(c * 8 + s + 1) * 128] for c in range(3)]
         for s in range(8)]

    def w1blk(r, part, c):
        m = (r * 2 + part) * 3 + c
        return w1s_ref[m * 128:(m + 1) * 128, :]

    def c1(r, s):
        acc = jnp.dot(X[s][0], w1blk(r, 0, 0), preferred_element_type=f32)
        for c in (1, 2):
            acc += jnp.dot(X[s][c], w1blk(r, 0, c), preferred_element_type=f32)
        for c in (0, 1, 2):
            acc += jnp.dot(X[s + 1][c], w1blk(r, 1, c), preferred_element_type=f32)
        return acc

    def pool1(a, b):
        m = jnp.maximum(a, b)
        return jnp.maximum(jnp.maximum(m[:, :128], m[:, 128:]) + b1, 0.0)

    p1 = []
    for s in range(7):
        pe = pool1(c1(0, s), c1(1, s))
        po = pool1(c1(2, s), c1(3, s))
        p1.append(jnp.concatenate([pe, po], axis=1).astype(bf16))

    p2 = []
    for q in range(5):
        ye = (jnp.dot(p1[q], w2_ref[:256, :], preferred_element_type=f32)
              + jnp.dot(p1[q + 1], w2_ref[256:512, :], preferred_element_type=f32)
              + jnp.dot(p1[q + 2][:, :128], w2_ref[512:, :], preferred_element_type=f32))
        yo = (jnp.dot(p1[q][:, 128:], w2_ref[:128, :], preferred_element_type=f32)
              + jnp.dot(p1[q + 1], w2_ref[128:384, :], preferred_element_type=f32)
              + jnp.dot(p1[q + 2], w2_ref[384:, :], preferred_element_type=f32))
        m2 = jnp.maximum(ye, yo)
        p2.append(jnp.maximum(jnp.maximum(m2[:, :128], m2[:, 128:]) + b2,
                              0.0).astype(bf16))

    f1in = jnp.concatenate(p2, axis=1)
    f1 = jnp.dot(f1in, wf1_ref[...], preferred_element_type=f32) + bias_ref[2:3, :]
    f1 = jnp.maximum(f1, 0.0).astype(bf16)
    f2 = jnp.dot(f1, wf2_ref[...], preferred_element_type=f32) + bias_ref[3:4, :]
    f2 = jnp.maximum(f2, 0.0).astype(bf16)
    f3 = jnp.dot(f2, wf3_ref[...], preferred_element_type=f32) + bias_ref[4:5, :]
    out_ref[...] = f3.astype(out_ref.dtype)


def _build_w1_blocks(w1):
    g = jnp.arange(4)[:, None]
    w = jnp.arange(32)[None, :]
    blocks = []
    for r in range(4):
        for part in range(2):
            for c in range(3):
                kh = g - r if part == 0 else g + 4 - r
                ok = jnp.broadcast_to((kh >= 0) & (kh <= 4), (4, 32))
                rows = (jnp.clip(kh, 0, 4) * 128 + w * 3 + c).reshape(128)
                blk = w1[rows, :] * ok.reshape(128, 1).astype(w1.dtype)
                blocks.append(blk)
    return jnp.concatenate(blocks, axis=0)


def kernel(x, w1, w2, wf1, wf2, wf3, biases):
    N = x.shape[0]
    bt = 256
    while bt > 8 and (N + bt - 1) // bt < 2:
        bt //= 2
    n_pad = (-N) % bt
    nt = (N + n_pad) // bt

    xv = x
    if n_pad:
        xv = jnp.pad(xv, ((0, n_pad), (0, 0), (0, 0), (0, 0)))
    xv = xv.reshape(nt, bt, 3072)

    w1s = _build_w1_blocks(w1)

    x_specs = [pl.BlockSpec((1, bt, 3072), lambda i: (i, 0, 0))]
    const_specs = [
        pl.BlockSpec((3072, 256), lambda i: (0, 0)),
        pl.BlockSpec((640, 256), lambda i: (0, 0)),
        pl.BlockSpec((640, 128), lambda i: (0, 0)),
        pl.BlockSpec((128, 128), lambda i: (0, 0)),
        pl.BlockSpec((128, 128), lambda i: (0, 0)),
        pl.BlockSpec((8, 128), lambda i: (0, 0)),
    ]

    out = pl.pallas_call(
        functools.partial(_fused_body, bt=bt),
        out_shape=jax.ShapeDtypeStruct((nt * bt, 128), jnp.float32),
        grid=(nt,),
        in_specs=x_specs + const_specs,
        out_specs=pl.BlockSpec((bt, 128), lambda i: (i, 0)),
        compiler_params=pltpu.CompilerParams(
            dimension_semantics=("parallel",),
            vmem_limit_bytes=64 * 1024 * 1024,
        ),
    )(xv, w1s, w2, wf1, wf2, wf3, biases)

    return out[:N, :10]


# speedup vs baseline: 1.7915x; 1.0318x over previous
import functools

import numpy as np

import jax
import jax.numpy as jnp
from jax.experimental import pallas as pl
from jax.experimental.pallas import tpu as pltpu


def _fused_body(xa_ref, xb_ref, w1s_ref, w2_ref, wf1_ref, wf2_ref, wf3_ref,
                bias_ref, out_ref, *, bt):
    f32 = jnp.float32
    bf16 = jnp.bfloat16
    b1 = bias_ref[0:1, :]
    b2 = bias_ref[1:2, :]

    def slab(c, s):
        m = c * 8 + s
        r, off = (xa_ref, m) if m < 12 else (xb_ref, m - 12)
        return r[0, :, off * 128:(off + 1) * 128]

    X = [[slab(c, s) for c in range(3)] for s in range(8)]

    def w1blk(r, part, c):
        m = (r * 2 + part) * 3 + c
        return w1s_ref[m * 128:(m + 1) * 128, :]

    def c1(r, s):
        acc = jnp.dot(X[s][0], w1blk(r, 0, 0), preferred_element_type=f32)
        for c in (1, 2):
            acc += jnp.dot(X[s][c], w1blk(r, 0, c), preferred_element_type=f32)
        for c in (0, 1, 2):
            acc += jnp.dot(X[s + 1][c], w1blk(r, 1, c), preferred_element_type=f32)
        return acc

    def pool1(a, b):
        m = jnp.maximum(a, b)
        return jnp.maximum(jnp.maximum(m[:, :128], m[:, 128:]) + b1, 0.0)

    p1 = []
    for s in range(7):
        pe = pool1(c1(0, s), c1(1, s))
        po = pool1(c1(2, s), c1(3, s))
        p1.append(jnp.concatenate([pe, po], axis=1).astype(bf16))

    p2 = []
    for q in range(5):
        ye = (jnp.dot(p1[q], w2_ref[:256, :], preferred_element_type=f32)
              + jnp.dot(p1[q + 1], w2_ref[256:512, :], preferred_element_type=f32)
              + jnp.dot(p1[q + 2][:, :128], w2_ref[512:, :], preferred_element_type=f32))
        yo = (jnp.dot(p1[q][:, 128:], w2_ref[:128, :], preferred_element_type=f32)
              + jnp.dot(p1[q + 1], w2_ref[128:384, :], preferred_element_type=f32)
              + jnp.dot(p1[q + 2], w2_ref[384:, :], preferred_element_type=f32))
        m2 = jnp.maximum(ye, yo)
        p2.append(jnp.maximum(jnp.maximum(m2[:, :128], m2[:, 128:]) + b2,
                              0.0).astype(bf16))

    f1in = jnp.concatenate(p2, axis=1)
    f1 = jnp.dot(f1in, wf1_ref[...], preferred_element_type=f32) + bias_ref[2:3, :]
    f1 = jnp.maximum(f1, 0.0).astype(bf16)
    f2 = jnp.dot(f1, wf2_ref[...], preferred_element_type=f32) + bias_ref[3:4, :]
    f2 = jnp.maximum(f2, 0.0).astype(bf16)
    f3 = jnp.dot(f2, wf3_ref[...], preferred_element_type=f32) + bias_ref[4:5, :]
    out_ref[...] = f3.astype(out_ref.dtype)


def _w1_block_rows():
    g = np.arange(4)[:, None]
    w = np.arange(32)[None, :]
    rows = []
    for r in range(4):
        for part in range(2):
            for c in range(3):
                kh = g - r if part == 0 else g + 4 - r
                ok = np.broadcast_to((kh >= 0) & (kh <= 4), (4, 32))
                idx = np.where(ok, np.clip(kh, 0, 4) * 128 + w * 3 + c, 640)
                rows.append(idx.reshape(128))
    return np.concatenate(rows)


_W1_ROWS = _w1_block_rows()


def kernel(x, w1, w2, wf1, wf2, wf3, biases):
    N = x.shape[0]
    bt = 256
    while bt > 8 and (N + bt - 1) // bt < 2:
        bt //= 2
    n_pad = (-N) % bt
    nt = (N + n_pad) // bt

    xv = x
    if n_pad:
        xv = jnp.pad(xv, ((0, n_pad), (0, 0), (0, 0), (0, 0)))
    xv = xv.reshape(nt, bt, 3072).astype(jnp.bfloat16)

    w1ext = jnp.concatenate([w1, jnp.zeros((1, 256), w1.dtype)], axis=0)
    w1s = w1ext[jnp.asarray(_W1_ROWS), :]

    specs = [
        pl.BlockSpec((1, bt, 1536), lambda i: (i, 0, 0)),
        pl.BlockSpec((1, bt, 1536), lambda i: (i, 0, 1)),
        pl.BlockSpec((3072, 256), lambda i: (0, 0)),
        pl.BlockSpec((640, 256), lambda i: (0, 0)),
        pl.BlockSpec((640, 128), lambda i: (0, 0)),
        pl.BlockSpec((128, 128), lambda i: (0, 0)),
        pl.BlockSpec((128, 128), lambda i: (0, 0)),
        pl.BlockSpec((8, 128), lambda i: (0, 0)),
    ]

    out = pl.pallas_call(
        functools.partial(_fused_body, bt=bt),
        out_shape=jax.ShapeDtypeStruct((nt * bt, 128), jnp.float32),
        grid=(nt,),
        in_specs=specs,
        out_specs=pl.BlockSpec((bt, 128), lambda i: (i, 0)),
        compiler_params=pltpu.CompilerParams(
            dimension_semantics=("parallel",),
            vmem_limit_bytes=64 * 1024 * 1024,
        ),
    )(xv, xv, w1s, w2, wf1, wf2, wf3, biases)

    return out[:N, :10]
